# baseline (speedup 1.0000x reference)
import functools

import numpy as np
import jax
import jax.numpy as jnp
from jax import lax
from jax.experimental import pallas as pl
from jax.experimental.pallas import tpu as pltpu
from jax.experimental.pallas import tpu_sc as plsc


def _positional_encoding(sequence_length, embedding_depth):
    half = embedding_depth / 2
    positions = np.arange(sequence_length)[:, np.newaxis]
    depths = np.arange(half)[np.newaxis, :] / half
    angle_rates = 1 / 10000 ** depths
    angle_rads = positions * angle_rates
    enc = np.concatenate([np.sin(angle_rads), np.cos(angle_rads)], axis=-1)
    return enc.astype(np.float32)


_B, _T, _D = 4096, 200, 64
_CH = 128
_NB = 2
_NW = 32
_NBLK = _B // _CH
_UNITS = _T * _NBLK
_UPW = _UNITS // _NW
_SLAB = 4
_NSLAB = _T // _SLAB
_USLAB = _SLAB * _NBLK
_LANES = 16


def _build():
    mesh = plsc.VectorSubcoreMesh(core_axis_name="c", subcore_axis_name="s")
    out_type = jax.ShapeDtypeStruct((_T, _D // 8, _NBLK, 8, _CH), jnp.float32)
    scratch = [
        pltpu.VMEM((_SLAB, _B), jnp.int32),
        pltpu.VMEM((_T, _D), jnp.float32),
        pltpu.VMEM((_NB, _CH), jnp.int32),
        pltpu.VMEM((_NB, _CH), jnp.int32),
    ]
    scratch += [pltpu.VMEM((_CH, 2 * _D), jnp.float32) for _ in range(_NB)]
    scratch += [pltpu.VMEM((_D, _CH), jnp.float32) for _ in range(_NB)]
    scratch += [pltpu.SemaphoreType.DMA] * (2 * _NB + 1)

    @functools.partial(pl.kernel, mesh=mesh, out_type=out_type,
                       scratch_types=scratch,
                       compiler_params=pltpu.CompilerParams(
                           use_tc_tiling_on_sc=True,
                           needs_layout_passes=False))
    def k(xt3, tp, pos, out, slab_v, pos_v, hidx, pbuf, *rest):
        gb = rest[0:_NB]
        ob = rest[_NB:2 * _NB]
        gsem = rest[2 * _NB:3 * _NB]
        wsem = rest[3 * _NB:4 * _NB]
        ssem = rest[4 * _NB]

        wid = lax.axis_index("s") * 2 + lax.axis_index("c")
        u0 = wid * _UPW

        lanes = lax.iota(jnp.int32, _LANES)

        pltpu.make_async_copy(pos, pos_v, ssem).start()
        pltpu.make_async_copy(pos, pos_v, ssem).wait()

        def stage_slab(u):
            s = u // _USLAB
            pltpu.make_async_copy(xt3.at[s], slab_v, ssem).start()
            pltpu.make_async_copy(xt3.at[s], slab_v, ssem).wait()

        def start_gather(u, b):
            tl = (u // _NBLK) % _SLAB
            blk = u % _NBLK
            for v in range(_CH // _LANES):
                sl = pl.ds(v * _LANES, _LANES)
                raw = slab_v[tl, pl.ds(blk * _CH + v * _LANES, _LANES)]
                hidx[b, sl] = lax.shift_right_logical(raw, 1)
                pbuf[b, sl] = (raw & 1) * _D
            pltpu.make_async_copy(tp.at[hidx.at[b]], gb[b], gsem[b]).start()

        def gather_wait(b):
            pltpu.make_async_copy(tp.at[hidx.at[b]], gb[b], gsem[b]).wait()

        def write_unit(u, b, do_wait):
            t = u // _NBLK
            blk = u % _NBLK
            for g in range(_D // 8):
                cp = pltpu.make_async_copy(
                    ob[b].at[pl.ds(g * 8, 8)], out.at[t, g, blk], wsem[b])
                if do_wait:
                    cp.wait()
                else:
                    cp.start()

        stage_slab(u0)
        for b in range(_NB):
            start_gather(u0 + b, b)

        def group(gi, carry):
            for b in range(_NB):
                u = u0 + gi * _NB + b
                t = u // _NBLK
                gather_wait(b)

                @pl.when(gi > 0)
                def _():
                    write_unit(u - _NB, b, True)

                rvecs = [lanes + lb * _LANES for lb in range(_CH // _LANES)]
                pv64 = [pbuf[b, pl.ds(lb * _LANES, _LANES)]
                        for lb in range(_CH // _LANES)]
                tsplat = lanes * 0 + t

                def col_body(c, cc):
                    csplat = lanes * 0 + c
                    pvreg = plsc.load_gather(pos_v, [tsplat, csplat])
                    for lb in range(_CH // _LANES):
                        colv = pv64[lb] + c
                        val = plsc.load_gather(gb[b], [rvecs[lb], colv])
                        osl = pl.ds(lb * _LANES, _LANES)
                        ob[b][c, osl] = val + pvreg
                    return cc

                lax.fori_loop(0, _D, col_body, 0)

                nu = u + _NB

                @pl.when(jnp.logical_and(
                    nu < u0 + _UPW, (nu // _USLAB) != (u // _USLAB)))
                def _():
                    stage_slab(nu)

                @pl.when(nu < u0 + _UPW)
                def _():
                    start_gather(nu, b)

                write_unit(u, b, False)
            return carry

        lax.fori_loop(0, _UPW // _NB, group, 0)
        for b in range(_NB):
            write_unit(u0 + _UPW - _NB + b, b, True)

    return k


_KERNEL = _build()


def kernel(x, table):
    xt3 = x.astype(jnp.int32).T.reshape(_NSLAB, _SLAB, _B)
    tp = table.reshape(500000, 2 * _D)
    pos = jnp.asarray(_positional_encoding(_T, _D))
    o5 = _KERNEL(xt3, tp, pos)
    return o5.transpose((2, 4, 0, 1, 3)).reshape(_B, _T, _D)


# speedup vs baseline: 1.5833x; 1.5833x over previous
import functools

import numpy as np
import jax
import jax.numpy as jnp
from jax import lax
from jax.experimental import pallas as pl
from jax.experimental.pallas import tpu as pltpu
from jax.experimental.pallas import tpu_sc as plsc


def _positional_encoding(sequence_length, embedding_depth):
    half = embedding_depth / 2
    positions = np.arange(sequence_length)[:, np.newaxis]
    depths = np.arange(half)[np.newaxis, :] / half
    angle_rates = 1 / 10000 ** depths
    angle_rads = positions * angle_rates
    enc = np.concatenate([np.sin(angle_rads), np.cos(angle_rads)], axis=-1)
    return enc.astype(np.float32)


_B, _T, _D = 4096, 200, 64
_CHUNK = 100
_NB = 4
_NW = 32
_NCHUNK = (_B * _T) // (_CHUNK * _NW)
_GROUPS = _NCHUNK // _NB
_LANES = 16


def _build():
    mesh = plsc.VectorSubcoreMesh(core_axis_name="c", subcore_axis_name="s")
    out_type = jax.ShapeDtypeStruct((_B * _T * _D,), jnp.float32)
    scratch = [
        pltpu.VMEM((_NCHUNK, _CHUNK), jnp.int32),
        pltpu.VMEM((_T, _D), jnp.float32),
    ]
    scratch += [pltpu.VMEM((_CHUNK, _D), jnp.float32) for _ in range(_NB)]
    scratch += [pltpu.VMEM((_CHUNK * _D,), jnp.float32) for _ in range(_NB)]
    scratch += [pltpu.SemaphoreType.DMA] * (2 * _NB)

    @functools.partial(pl.kernel, mesh=mesh, out_type=out_type,
                       scratch_types=scratch,
                       compiler_params=pltpu.CompilerParams(
                           use_tc_tiling_on_sc=False))
    def k(xr, table, pos, out, idx_v, pos_v, *rest):
        gb = rest[0:_NB]
        ob = rest[_NB:2 * _NB]
        gsem = rest[2 * _NB:3 * _NB]
        wsem = rest[3 * _NB:4 * _NB]

        wid = lax.axis_index("s") * 2 + lax.axis_index("c")
        row0 = wid * _NCHUNK
        out0 = wid * _NCHUNK * _CHUNK * _D

        pltpu.sync_copy(xr.at[pl.ds(row0, _NCHUNK)], idx_v)
        pltpu.sync_copy(pos, pos_v)

        def gather(j, b):
            return pltpu.make_async_copy(table.at[idx_v.at[j]], gb[b], gsem[b])

        def write(j, b):
            dst = out.at[pl.ds(out0 + j * _CHUNK * _D, _CHUNK * _D)]
            return pltpu.make_async_copy(ob[b], dst, wsem[b])

        for b in range(_NB):
            gather(b, b).start()

        def group(gi, carry):
            for b in range(_NB):
                j = gi * _NB + b
                gather(j, b).wait()

                @pl.when(gi > 0)
                def _():
                    write(j - _NB, b).wait()

                prow0 = (j % 2) * _CHUNK

                def add_row(r, c):
                    pr = prow0 + r
                    for d in range(_D // _LANES):
                        sl = pl.ds(d * _LANES, _LANES)
                        osl = pl.ds(r * _D + d * _LANES, _LANES)
                        ob[b][osl] = gb[b][r, sl] + pos_v[pr, sl]
                    return c

                lax.fori_loop(0, _CHUNK, add_row, 0)
                write(j, b).start()

                @pl.when(j + _NB < _NCHUNK)
                def _():
                    gather(j + _NB, b).start()
            return carry

        lax.fori_loop(0, _GROUPS, group, 0)
        for b in range(_NB):
            write(_NCHUNK - _NB + b, b).wait()

    return k


_KERNEL = _build()


def kernel(x, table):
    xr = x.reshape(_B * _T // _CHUNK, _CHUNK).astype(jnp.int32)
    pos = jnp.asarray(_positional_encoding(_T, _D))
    out = _KERNEL(xr, table, pos)
    return out.reshape(_B, _T, _D)
